# baseline (device time: 300418 ns/iter reference)
import jax
import jax.numpy as jnp
from jax import lax
from jax.experimental import pallas as pl
from jax.experimental.pallas import tpu as pltpu


N_CHUNK = 16
GATHER_WINDOW = 32
GATHER_UNROLL = 4


def _gather(idx, scale, E):
    t = idx.shape[0]
    _, d = E.shape

    def body(idx_ref, scale_ref, E_ref, out_ref, fstage, sems):
        w = GATHER_WINDOW

        def start(i):
            pltpu.make_async_copy(
                E_ref.at[pl.ds(idx_ref[i], 1), :],
                fstage.at[pl.ds(i, 1), :],
                sems.at[i % w],
            ).start()

        def wait(k):
            pltpu.make_async_copy(
                E_ref.at[pl.ds(0, 1), :],
                fstage.at[pl.ds(0, 1), :],
                sems.at[k],
            ).wait()

        for i in range(w):
            start(i)

        def step(j, _):
            i = w + j * GATHER_UNROLL
            for u in range(GATHER_UNROLL):
                wait((i + u) % w)
                start(i + u)
            return 0

        lax.fori_loop(0, (t - w) // GATHER_UNROLL, step, 0)
        for k in range(w):
            wait(k)
        out_ref[...] = (fstage[...] * scale_ref[...]).astype(jnp.bfloat16)

    return pl.pallas_call(
        body,
        out_shape=jax.ShapeDtypeStruct((t, d), jnp.bfloat16),
        in_specs=[
            pl.BlockSpec(memory_space=pltpu.SMEM),
            pl.BlockSpec(memory_space=pltpu.VMEM),
            pl.BlockSpec(memory_space=pltpu.MemorySpace.HBM),
        ],
        out_specs=pl.BlockSpec(memory_space=pltpu.VMEM),
        scratch_shapes=[
            pltpu.VMEM((t, d), jnp.float32),
            pltpu.SemaphoreType.DMA((GATHER_WINDOW,)),
        ],
        compiler_params=pltpu.CompilerParams(
            vmem_limit_bytes=56 * 1024 * 1024
        ),
    )(idx, scale, E)


def _allreduce_y(partial):
    t, d = partial.shape
    half = t // 2
    rows = half // N_CHUNK

    def body(p_ref, out_ref, ybuf, xbuf, ysend, yrecv, xsend, xrecv):
        my_x = lax.axis_index("x")
        my_y = lax.axis_index("y")
        my_z = lax.axis_index("z")
        ynbr = (my_x, 1 - my_y, my_z)
        xnbr = (1 - my_x, my_y, my_z)

        barrier = pltpu.get_barrier_semaphore()
        for nbr in (ynbr, xnbr):
            pl.semaphore_signal(
                barrier, inc=1, device_id=nbr, device_id_type=pl.DeviceIdType.MESH
            )
        pl.semaphore_wait(barrier, 2)

        my_off = my_x * half
        other_off = (1 - my_x) * half

        y_rdmas = []
        for c in range(N_CHUNK):
            r0 = c * rows
            rd = pltpu.make_async_remote_copy(
                src_ref=p_ref.at[pl.ds(my_off + r0, rows), :],
                dst_ref=ybuf.at[pl.ds(r0, rows), :],
                send_sem=ysend.at[c],
                recv_sem=yrecv.at[c],
                device_id=ynbr,
                device_id_type=pl.DeviceIdType.MESH,
            )
            rd.start()
            y_rdmas.append(rd)

        x_rdmas = []
        for c in range(N_CHUNK):
            r0 = c * rows
            y_rdmas[c].wait()
            rd = pltpu.make_async_remote_copy(
                src_ref=ybuf.at[pl.ds(r0, rows), :],
                dst_ref=xbuf.at[pl.ds(r0, rows), :],
                send_sem=xsend.at[c],
                recv_sem=xrecv.at[c],
                device_id=xnbr,
                device_id_type=pl.DeviceIdType.MESH,
            )
            rd.start()
            x_rdmas.append(rd)
            out_ref[pl.ds(my_off + r0, rows), :] = (
                p_ref[pl.ds(my_off + r0, rows), :] + ybuf[pl.ds(r0, rows), :]
            )

        for c in range(N_CHUNK):
            r0 = c * rows
            x_rdmas[c].wait_recv()
            out_ref[pl.ds(other_off + r0, rows), :] = (
                p_ref[pl.ds(other_off + r0, rows), :] + xbuf[pl.ds(r0, rows), :]
            )

        for c in range(N_CHUNK):
            x_rdmas[c].wait_send()

    return pl.pallas_call(
        body,
        out_shape=jax.ShapeDtypeStruct((t, d), jnp.bfloat16),
        in_specs=[pl.BlockSpec(memory_space=pltpu.VMEM)],
        out_specs=pl.BlockSpec(memory_space=pltpu.VMEM),
        input_output_aliases={0: 0},
        scratch_shapes=[
            pltpu.VMEM((half, d), jnp.bfloat16),
            pltpu.VMEM((half, d), jnp.bfloat16),
            pltpu.SemaphoreType.DMA((N_CHUNK,)),
            pltpu.SemaphoreType.DMA((N_CHUNK,)),
            pltpu.SemaphoreType.DMA((N_CHUNK,)),
            pltpu.SemaphoreType.DMA((N_CHUNK,)),
        ],
        compiler_params=pltpu.CompilerParams(
            collective_id=0, vmem_limit_bytes=56 * 1024 * 1024
        ),
    )(partial)


def kernel(ids, E):
    v_per = E.shape[0]
    my_y = lax.axis_index("y")
    local = ids - my_y * v_per
    mask = (local >= 0) & (local < v_per)
    idx = jnp.clip(local, 0, v_per - 1)
    scale = mask.astype(jnp.float32)[:, None]
    partial = _gather(idx, scale, E)
    return _allreduce_y(partial).astype(jnp.float32)


# device time: 143643 ns/iter; 2.0914x vs baseline; 2.0914x over previous
import jax
import jax.numpy as jnp
from jax import lax
from jax.experimental import pallas as pl
from jax.experimental.pallas import tpu as pltpu

N_CHUNK = 16
W = 64
RING = 4


def _fused(idx, scale, E):
    t = idx.shape[0]
    _, d = E.shape
    half = t // 2
    rows = half // N_CHUNK

    def body(
        idx_ref, scale_ref, E_ref, out_ref, fring, ybuf, gsems, ysend, yrecv, xsend, xrecv
    ):
        my_x = lax.axis_index("x")
        my_y = lax.axis_index("y")
        my_z = lax.axis_index("z")
        ynbr = (my_x, 1 - my_y, my_z)
        xnbr = (1 - my_x, my_y, my_z)
        my_off = my_x * half

        barrier = pltpu.get_barrier_semaphore()
        for nbr in (ynbr, xnbr):
            pl.semaphore_signal(
                barrier, inc=1, device_id=nbr, device_id_type=pl.DeviceIdType.MESH
            )
        pl.semaphore_wait(barrier, 2)

        def g_start(i, j, slot):
            pltpu.make_async_copy(
                E_ref.at[pl.ds(idx_ref[my_off + i], 1), :],
                fring.at[slot].at[pl.ds(j, 1), :],
                gsems.at[lax.rem(i, W)],
            ).start()

        def g_wait(k):
            pltpu.make_async_copy(
                E_ref.at[pl.ds(0, 1), :],
                fring.at[0].at[pl.ds(0, 1), :],
                gsems.at[k],
            ).wait()

        def issue_segment(s):
            slot = s % RING
            base = s * rows
            if s == 0:
                def b0(k, _):
                    for u in range(4):
                        i = k * 4 + u
                        g_start(i, i, slot)
                    return 0

                lax.fori_loop(0, W // 4, b0, 0)

                def b1(k, _):
                    for u in range(4):
                        i = W + k * 4 + u
                        g_wait(lax.rem(i, W))
                        g_start(i, i, slot)
                    return 0

                lax.fori_loop(0, (rows - W) // 4, b1, 0)
            else:
                def b(k, _):
                    for u in range(4):
                        i = base + k * 4 + u
                        g_wait(lax.rem(i, W))
                        g_start(i, i - base, slot)
                    return 0

                lax.fori_loop(0, rows // 4, b, 0)

        def convert(cs):
            g0 = my_off + cs * rows
            out_ref[pl.ds(g0, rows), :] = (
                fring[cs % RING] * scale_ref[pl.ds(g0, rows), :]
            ).astype(jnp.bfloat16)

        y_rdmas = [None] * N_CHUNK
        x_rdmas = [None] * N_CHUNK

        def y_send(c):
            rd = pltpu.make_async_remote_copy(
                src_ref=out_ref.at[pl.ds(my_off + c * rows, rows), :],
                dst_ref=ybuf.at[pl.ds(c * rows, rows), :],
                send_sem=ysend.at[c],
                recv_sem=yrecv.at[c],
                device_id=ynbr,
                device_id_type=pl.DeviceIdType.MESH,
            )
            rd.start()
            y_rdmas[c] = rd

        def y_process(c):
            y_rdmas[c].wait()
            g0 = my_off + c * rows
            out_ref[pl.ds(g0, rows), :] = (
                out_ref[pl.ds(g0, rows), :] + ybuf[pl.ds(c * rows, rows), :]
            )
            rd = pltpu.make_async_remote_copy(
                src_ref=out_ref.at[pl.ds(g0, rows), :],
                dst_ref=out_ref.at[pl.ds(g0, rows), :],
                send_sem=xsend.at[c],
                recv_sem=xrecv.at[c],
                device_id=xnbr,
                device_id_type=pl.DeviceIdType.MESH,
            )
            rd.start()
            x_rdmas[c] = rd

        for s in range(N_CHUNK):
            issue_segment(s)
            if s >= 1:
                convert(s - 1)
                y_send(s - 1)
            if s >= 3:
                y_process(s - 3)

        for k in range(W):
            g_wait(k)
        convert(N_CHUNK - 1)
        y_send(N_CHUNK - 1)
        for c in range(N_CHUNK - 3, N_CHUNK):
            y_process(c)
        for c in range(N_CHUNK):
            x_rdmas[c].wait_recv()
        for c in range(N_CHUNK):
            x_rdmas[c].wait_send()

    return pl.pallas_call(
        body,
        out_shape=jax.ShapeDtypeStruct((t, d), jnp.bfloat16),
        in_specs=[
            pl.BlockSpec(memory_space=pltpu.SMEM),
            pl.BlockSpec(memory_space=pltpu.VMEM),
            pl.BlockSpec(memory_space=pltpu.MemorySpace.HBM),
        ],
        out_specs=pl.BlockSpec(memory_space=pltpu.VMEM),
        scratch_shapes=[
            pltpu.VMEM((RING, rows, d), jnp.float32),
            pltpu.VMEM((half, d), jnp.bfloat16),
            pltpu.SemaphoreType.DMA((W,)),
            pltpu.SemaphoreType.DMA((N_CHUNK,)),
            pltpu.SemaphoreType.DMA((N_CHUNK,)),
            pltpu.SemaphoreType.DMA((N_CHUNK,)),
            pltpu.SemaphoreType.DMA((N_CHUNK,)),
        ],
        compiler_params=pltpu.CompilerParams(
            collective_id=0, vmem_limit_bytes=56 * 1024 * 1024
        ),
    )(idx, scale, E)


def kernel(ids, E):
    v_per = E.shape[0]
    my_y = lax.axis_index("y")
    local = ids - my_y * v_per
    mask = (local >= 0) & (local < v_per)
    idx = jnp.clip(local, 0, v_per - 1)
    scale = mask.astype(jnp.float32)[:, None]
    return _fused(idx, scale, E).astype(jnp.float32)


# device time: 137752 ns/iter; 2.1809x vs baseline; 1.0428x over previous
import jax
import jax.numpy as jnp
from jax import lax
from jax.experimental import pallas as pl
from jax.experimental.pallas import tpu as pltpu

N_CHUNK = 16
W = 128
RING = 4


def _fused(idx, scale, E):
    t = idx.shape[0]
    _, d = E.shape
    half = t // 2
    rows = half // N_CHUNK

    def body(
        idx_ref, scale_ref, E_ref, out_ref, fring, ybuf, gsems, ysend, yrecv, xsend, xrecv
    ):
        my_x = lax.axis_index("x")
        my_y = lax.axis_index("y")
        my_z = lax.axis_index("z")
        ynbr = (my_x, 1 - my_y, my_z)
        xnbr = (1 - my_x, my_y, my_z)
        my_off = my_x * half

        barrier = pltpu.get_barrier_semaphore()
        for nbr in (ynbr, xnbr):
            pl.semaphore_signal(
                barrier, inc=1, device_id=nbr, device_id_type=pl.DeviceIdType.MESH
            )
        pl.semaphore_wait(barrier, 2)

        def g_start(i, j, slot):
            pltpu.make_async_copy(
                E_ref.at[pl.ds(idx_ref[my_off + i], 1), :],
                fring.at[slot].at[pl.ds(j, 1), :],
                gsems.at[lax.rem(i, W)],
            ).start()

        def g_wait(k):
            pltpu.make_async_copy(
                E_ref.at[pl.ds(0, 1), :],
                fring.at[0].at[pl.ds(0, 1), :],
                gsems.at[k],
            ).wait()

        def issue_segment(s):
            slot = s % RING
            base = s * rows
            if s == 0:
                def b0(k, _):
                    for u in range(4):
                        i = k * 4 + u
                        g_start(i, i, slot)
                    return 0

                lax.fori_loop(0, W // 4, b0, 0)

                def b1(k, _):
                    for u in range(4):
                        i = W + k * 4 + u
                        g_wait(lax.rem(i, W))
                        g_start(i, i, slot)
                    return 0

                lax.fori_loop(0, (rows - W) // 4, b1, 0)
            else:
                def b(k, _):
                    for u in range(4):
                        i = base + k * 4 + u
                        g_wait(lax.rem(i, W))
                        g_start(i, i - base, slot)
                    return 0

                lax.fori_loop(0, rows // 4, b, 0)

        def convert(cs):
            g0 = my_off + cs * rows
            out_ref[pl.ds(g0, rows), :] = (
                fring[cs % RING] * scale_ref[pl.ds(g0, rows), :]
            ).astype(jnp.bfloat16)

        y_rdmas = [None] * N_CHUNK
        x_rdmas = [None] * N_CHUNK

        def y_send(c):
            rd = pltpu.make_async_remote_copy(
                src_ref=out_ref.at[pl.ds(my_off + c * rows, rows), :],
                dst_ref=ybuf.at[pl.ds(c * rows, rows), :],
                send_sem=ysend.at[c],
                recv_sem=yrecv.at[c],
                device_id=ynbr,
                device_id_type=pl.DeviceIdType.MESH,
            )
            rd.start()
            y_rdmas[c] = rd

        def y_process(c):
            y_rdmas[c].wait()
            g0 = my_off + c * rows
            out_ref[pl.ds(g0, rows), :] = (
                out_ref[pl.ds(g0, rows), :] + ybuf[pl.ds(c * rows, rows), :]
            )
            rd = pltpu.make_async_remote_copy(
                src_ref=out_ref.at[pl.ds(g0, rows), :],
                dst_ref=out_ref.at[pl.ds(g0, rows), :],
                send_sem=xsend.at[c],
                recv_sem=xrecv.at[c],
                device_id=xnbr,
                device_id_type=pl.DeviceIdType.MESH,
            )
            rd.start()
            x_rdmas[c] = rd

        for s in range(N_CHUNK):
            issue_segment(s)
            if s >= 1:
                convert(s - 1)
                y_send(s - 1)
            if s >= 2:
                y_process(s - 2)

        for k in range(W):
            g_wait(k)
        convert(N_CHUNK - 1)
        y_send(N_CHUNK - 1)
        for c in range(N_CHUNK - 2, N_CHUNK):
            y_process(c)
        for c in range(N_CHUNK):
            x_rdmas[c].wait_recv()
        for c in range(N_CHUNK):
            x_rdmas[c].wait_send()

    return pl.pallas_call(
        body,
        out_shape=jax.ShapeDtypeStruct((t, d), jnp.bfloat16),
        in_specs=[
            pl.BlockSpec(memory_space=pltpu.SMEM),
            pl.BlockSpec(memory_space=pltpu.VMEM),
            pl.BlockSpec(memory_space=pltpu.MemorySpace.HBM),
        ],
        out_specs=pl.BlockSpec(memory_space=pltpu.VMEM),
        scratch_shapes=[
            pltpu.VMEM((RING, rows, d), jnp.float32),
            pltpu.VMEM((half, d), jnp.bfloat16),
            pltpu.SemaphoreType.DMA((W,)),
            pltpu.SemaphoreType.DMA((N_CHUNK,)),
            pltpu.SemaphoreType.DMA((N_CHUNK,)),
            pltpu.SemaphoreType.DMA((N_CHUNK,)),
            pltpu.SemaphoreType.DMA((N_CHUNK,)),
        ],
        compiler_params=pltpu.CompilerParams(
            collective_id=0, vmem_limit_bytes=56 * 1024 * 1024
        ),
    )(idx, scale, E)


def kernel(ids, E):
    v_per = E.shape[0]
    my_y = lax.axis_index("y")
    local = ids - my_y * v_per
    mask = (local >= 0) & (local < v_per)
    idx = jnp.clip(local, 0, v_per - 1)
    scale = mask.astype(jnp.float32)[:, None]
    return _fused(idx, scale, E).astype(jnp.float32)


# device time: 131791 ns/iter; 2.2795x vs baseline; 1.0452x over previous
import jax
import jax.numpy as jnp
from jax import lax
from jax.experimental import pallas as pl
from jax.experimental.pallas import tpu as pltpu

N_CHUNK = 32
W = 64
RING = 4


def _fused(idx, scale, E):
    t = idx.shape[0]
    _, d = E.shape
    half = t // 2
    rows = half // N_CHUNK

    def body(
        idx_ref, scale_ref, E_ref, out_ref, fring, ybuf, gsems, ysend, yrecv, xsend, xrecv
    ):
        my_x = lax.axis_index("x")
        my_y = lax.axis_index("y")
        my_z = lax.axis_index("z")
        ynbr = (my_x, 1 - my_y, my_z)
        xnbr = (1 - my_x, my_y, my_z)
        my_off = my_x * half

        barrier = pltpu.get_barrier_semaphore()
        for nbr in (ynbr, xnbr):
            pl.semaphore_signal(
                barrier, inc=1, device_id=nbr, device_id_type=pl.DeviceIdType.MESH
            )
        pl.semaphore_wait(barrier, 2)

        def g_start(i, j, slot):
            pltpu.make_async_copy(
                E_ref.at[pl.ds(idx_ref[my_off + i], 1), :],
                fring.at[slot].at[pl.ds(j, 1), :],
                gsems.at[lax.rem(i, W)],
            ).start()

        def g_wait(k):
            pltpu.make_async_copy(
                E_ref.at[pl.ds(0, 1), :],
                fring.at[0].at[pl.ds(0, 1), :],
                gsems.at[k],
            ).wait()

        def issue_segment(s):
            slot = s % RING
            base = s * rows
            if s == 0:
                def b0(k, _):
                    for u in range(4):
                        i = k * 4 + u
                        g_start(i, i, slot)
                    return 0

                lax.fori_loop(0, W // 4, b0, 0)

                def b1(k, _):
                    for u in range(4):
                        i = W + k * 4 + u
                        g_wait(lax.rem(i, W))
                        g_start(i, i, slot)
                    return 0

                lax.fori_loop(0, (rows - W) // 4, b1, 0)
            else:
                def b(k, _):
                    for u in range(4):
                        i = base + k * 4 + u
                        g_wait(lax.rem(i, W))
                        g_start(i, i - base, slot)
                    return 0

                lax.fori_loop(0, rows // 4, b, 0)

        def convert(cs):
            g0 = my_off + cs * rows
            out_ref[pl.ds(g0, rows), :] = (
                fring[cs % RING] * scale_ref[pl.ds(g0, rows), :]
            ).astype(jnp.bfloat16)

        y_rdmas = [None] * N_CHUNK
        x_rdmas = [None] * N_CHUNK

        def y_send(c):
            rd = pltpu.make_async_remote_copy(
                src_ref=out_ref.at[pl.ds(my_off + c * rows, rows), :],
                dst_ref=ybuf.at[pl.ds(c * rows, rows), :],
                send_sem=ysend.at[c],
                recv_sem=yrecv.at[c],
                device_id=ynbr,
                device_id_type=pl.DeviceIdType.MESH,
            )
            rd.start()
            y_rdmas[c] = rd

        def y_process(c):
            y_rdmas[c].wait()
            g0 = my_off + c * rows
            out_ref[pl.ds(g0, rows), :] = (
                out_ref[pl.ds(g0, rows), :] + ybuf[pl.ds(c * rows, rows), :]
            )
            rd = pltpu.make_async_remote_copy(
                src_ref=out_ref.at[pl.ds(g0, rows), :],
                dst_ref=out_ref.at[pl.ds(g0, rows), :],
                send_sem=xsend.at[c],
                recv_sem=xrecv.at[c],
                device_id=xnbr,
                device_id_type=pl.DeviceIdType.MESH,
            )
            rd.start()
            x_rdmas[c] = rd

        for s in range(N_CHUNK):
            issue_segment(s)
            if s >= 1:
                convert(s - 1)
                y_send(s - 1)
            if s >= 2:
                y_process(s - 2)

        for k in range(W):
            g_wait(k)
        convert(N_CHUNK - 1)
        y_send(N_CHUNK - 1)
        for c in range(N_CHUNK - 2, N_CHUNK):
            y_process(c)
        for c in range(N_CHUNK):
            x_rdmas[c].wait_recv()
        for c in range(N_CHUNK):
            x_rdmas[c].wait_send()

    return pl.pallas_call(
        body,
        out_shape=jax.ShapeDtypeStruct((t, d), jnp.bfloat16),
        in_specs=[
            pl.BlockSpec(memory_space=pltpu.SMEM),
            pl.BlockSpec(memory_space=pltpu.VMEM),
            pl.BlockSpec(memory_space=pltpu.MemorySpace.HBM),
        ],
        out_specs=pl.BlockSpec(memory_space=pltpu.VMEM),
        scratch_shapes=[
            pltpu.VMEM((RING, rows, d), jnp.float32),
            pltpu.VMEM((half, d), jnp.bfloat16),
            pltpu.SemaphoreType.DMA((W,)),
            pltpu.SemaphoreType.DMA((N_CHUNK,)),
            pltpu.SemaphoreType.DMA((N_CHUNK,)),
            pltpu.SemaphoreType.DMA((N_CHUNK,)),
            pltpu.SemaphoreType.DMA((N_CHUNK,)),
        ],
        compiler_params=pltpu.CompilerParams(
            collective_id=0, vmem_limit_bytes=56 * 1024 * 1024
        ),
    )(idx, scale, E)


def kernel(ids, E):
    v_per = E.shape[0]
    my_y = lax.axis_index("y")
    local = ids - my_y * v_per
    mask = (local >= 0) & (local < v_per)
    idx = jnp.clip(local, 0, v_per - 1)
    scale = mask.astype(jnp.float32)[:, None]
    return _fused(idx, scale, E)


# device time: 92102 ns/iter; 3.2618x vs baseline; 1.4309x over previous
import os

import jax
import jax.numpy as jnp
from jax import lax
from jax.experimental import pallas as pl
from jax.experimental.pallas import tpu as pltpu

NOCOMM = bool(int(os.environ.get("NOCOMM", "0")))
N_CHUNK = 32
W = 64
RING = 4


def _fused(idx, scale, E):
    t = idx.shape[0]
    _, d = E.shape
    half = t // 2
    rows = half // N_CHUNK

    def body(
        idx_ref, scale_ref, E_ref, out_ref, fring, ybuf, gsems, ysend, yrecv, xsend, xrecv
    ):
        my_x = lax.axis_index("x")
        my_y = lax.axis_index("y")
        my_z = lax.axis_index("z")
        ynbr = (my_x, 1 - my_y, my_z)
        xnbr = (1 - my_x, my_y, my_z)
        my_off = my_x * half

        if not NOCOMM:
            barrier = pltpu.get_barrier_semaphore()
            for nbr in (ynbr, xnbr):
                pl.semaphore_signal(
                    barrier, inc=1, device_id=nbr, device_id_type=pl.DeviceIdType.MESH
                )
            pl.semaphore_wait(barrier, 2)

        def g_start(i, j, slot):
            pltpu.make_async_copy(
                E_ref.at[pl.ds(idx_ref[my_off + i], 1), :],
                fring.at[slot].at[pl.ds(j, 1), :],
                gsems.at[lax.rem(i, W)],
            ).start()

        def g_wait(k):
            pltpu.make_async_copy(
                E_ref.at[pl.ds(0, 1), :],
                fring.at[0].at[pl.ds(0, 1), :],
                gsems.at[k],
            ).wait()

        def issue_segment(s):
            slot = s % RING
            base = s * rows
            if s == 0:
                def b0(k, _):
                    for u in range(4):
                        i = k * 4 + u
                        g_start(i, i, slot)
                    return 0

                lax.fori_loop(0, W // 4, b0, 0)

                def b1(k, _):
                    for u in range(4):
                        i = W + k * 4 + u
                        g_wait(lax.rem(i, W))
                        g_start(i, i, slot)
                    return 0

                lax.fori_loop(0, (rows - W) // 4, b1, 0)
            else:
                def b(k, _):
                    for u in range(4):
                        i = base + k * 4 + u
                        g_wait(lax.rem(i, W))
                        g_start(i, i - base, slot)
                    return 0

                lax.fori_loop(0, rows // 4, b, 0)

        def convert(cs):
            g0 = my_off + cs * rows
            out_ref[pl.ds(g0, rows), :] = (
                fring[cs % RING] * scale_ref[pl.ds(g0, rows), :]
            ).astype(jnp.bfloat16)

        y_rdmas = [None] * N_CHUNK
        x_rdmas = [None] * N_CHUNK

        def y_send(c):
            rd = pltpu.make_async_remote_copy(
                src_ref=out_ref.at[pl.ds(my_off + c * rows, rows), :],
                dst_ref=ybuf.at[pl.ds(c * rows, rows), :],
                send_sem=ysend.at[c],
                recv_sem=yrecv.at[c],
                device_id=ynbr,
                device_id_type=pl.DeviceIdType.MESH,
            )
            rd.start()
            y_rdmas[c] = rd

        def y_process(c):
            y_rdmas[c].wait()
            g0 = my_off + c * rows
            out_ref[pl.ds(g0, rows), :] = (
                out_ref[pl.ds(g0, rows), :] + ybuf[pl.ds(c * rows, rows), :]
            )
            rd = pltpu.make_async_remote_copy(
                src_ref=out_ref.at[pl.ds(g0, rows), :],
                dst_ref=out_ref.at[pl.ds(g0, rows), :],
                send_sem=xsend.at[c],
                recv_sem=xrecv.at[c],
                device_id=xnbr,
                device_id_type=pl.DeviceIdType.MESH,
            )
            rd.start()
            x_rdmas[c] = rd

        for s in range(N_CHUNK):
            issue_segment(s)
            if s >= 1:
                convert(s - 1)
                if not NOCOMM:
                    y_send(s - 1)
            if s >= 2 and not NOCOMM:
                y_process(s - 2)

        for k in range(W):
            g_wait(k)
        convert(N_CHUNK - 1)
        if not NOCOMM:
            y_send(N_CHUNK - 1)
            for c in range(N_CHUNK - 2, N_CHUNK):
                y_process(c)
            for c in range(N_CHUNK):
                x_rdmas[c].wait_recv()
            for c in range(N_CHUNK):
                x_rdmas[c].wait_send()

    return pl.pallas_call(
        body,
        out_shape=jax.ShapeDtypeStruct((t, d), jnp.bfloat16),
        in_specs=[
            pl.BlockSpec(memory_space=pltpu.SMEM),
            pl.BlockSpec(memory_space=pltpu.VMEM),
            pl.BlockSpec(memory_space=pltpu.MemorySpace.HBM),
        ],
        out_specs=pl.BlockSpec(memory_space=pltpu.VMEM),
        scratch_shapes=[
            pltpu.VMEM((RING, rows, d), jnp.float32),
            pltpu.VMEM((half, d), jnp.bfloat16),
            pltpu.SemaphoreType.DMA((W,)),
            pltpu.SemaphoreType.DMA((N_CHUNK,)),
            pltpu.SemaphoreType.DMA((N_CHUNK,)),
            pltpu.SemaphoreType.DMA((N_CHUNK,)),
            pltpu.SemaphoreType.DMA((N_CHUNK,)),
        ],
        compiler_params=pltpu.CompilerParams(
            collective_id=None if NOCOMM else 0,
            vmem_limit_bytes=56 * 1024 * 1024,
        ),
    )(idx, scale, E)


def kernel(ids, E):
    v_per = E.shape[0]
    my_y = lax.axis_index("y")
    local = ids - my_y * v_per
    mask = (local >= 0) & (local < v_per)
    idx = jnp.clip(local, 0, v_per - 1)
    scale = mask.astype(jnp.float32)[:, None]
    return _fused(idx, scale, E)


# device time: 91460 ns/iter; 3.2847x vs baseline; 1.0070x over previous
import os

import jax
import jax.numpy as jnp
from jax import lax
from jax.experimental import pallas as pl
from jax.experimental.pallas import tpu as pltpu

NOCOMM = bool(int(os.environ.get("NOCOMM", "0")))
N_CHUNK = 32
RING = 4
GATHER_UNROLL = 8


def _fused(idx, scale, E):
    t = idx.shape[0]
    _, d = E.shape
    half = t // 2
    rows = half // N_CHUNK

    def body(
        idx_ref, scale_ref, E_ref, out_ref, fring, ybuf, gsems, ysend, yrecv, xsend, xrecv
    ):
        my_x = lax.axis_index("x")
        my_y = lax.axis_index("y")
        my_z = lax.axis_index("z")
        ynbr = (my_x, 1 - my_y, my_z)
        xnbr = (1 - my_x, my_y, my_z)
        my_off = my_x * half

        if not NOCOMM:
            barrier = pltpu.get_barrier_semaphore()
            for nbr in (ynbr, xnbr):
                pl.semaphore_signal(
                    barrier, inc=1, device_id=nbr, device_id_type=pl.DeviceIdType.MESH
                )
            pl.semaphore_wait(barrier, 2)

        def issue_segment(s):
            slot = s % RING
            base = s * rows

            def b(k, _):
                for u in range(GATHER_UNROLL):
                    j = k * GATHER_UNROLL + u
                    pltpu.make_async_copy(
                        E_ref.at[pl.ds(idx_ref[my_off + base + j], 1), :],
                        fring.at[slot].at[pl.ds(j, 1), :],
                        gsems.at[slot],
                    ).start()
                return 0

            lax.fori_loop(0, rows // GATHER_UNROLL, b, 0)

        def convert(cs):
            pltpu.make_async_copy(
                E_ref.at[pl.ds(0, rows), :],
                fring.at[cs % RING],
                gsems.at[cs % RING],
            ).wait()
            g0 = my_off + cs * rows
            out_ref[pl.ds(g0, rows), :] = (
                fring[cs % RING] * scale_ref[pl.ds(g0, rows), :]
            ).astype(jnp.bfloat16)

        y_rdmas = [None] * N_CHUNK
        x_rdmas = [None] * N_CHUNK

        def y_send(c):
            rd = pltpu.make_async_remote_copy(
                src_ref=out_ref.at[pl.ds(my_off + c * rows, rows), :],
                dst_ref=ybuf.at[pl.ds(c * rows, rows), :],
                send_sem=ysend.at[c],
                recv_sem=yrecv.at[c],
                device_id=ynbr,
                device_id_type=pl.DeviceIdType.MESH,
            )
            rd.start()
            y_rdmas[c] = rd

        def y_process(c):
            y_rdmas[c].wait()
            g0 = my_off + c * rows
            out_ref[pl.ds(g0, rows), :] = (
                out_ref[pl.ds(g0, rows), :] + ybuf[pl.ds(c * rows, rows), :]
            )
            rd = pltpu.make_async_remote_copy(
                src_ref=out_ref.at[pl.ds(g0, rows), :],
                dst_ref=out_ref.at[pl.ds(g0, rows), :],
                send_sem=xsend.at[c],
                recv_sem=xrecv.at[c],
                device_id=xnbr,
                device_id_type=pl.DeviceIdType.MESH,
            )
            rd.start()
            x_rdmas[c] = rd

        for s in range(N_CHUNK):
            issue_segment(s)
            if s >= 1:
                convert(s - 1)
                if not NOCOMM:
                    y_send(s - 1)
            if s >= 2 and not NOCOMM:
                y_process(s - 2)

        convert(N_CHUNK - 1)
        if not NOCOMM:
            y_send(N_CHUNK - 1)
            for c in range(N_CHUNK - 2, N_CHUNK):
                y_process(c)
            for c in range(N_CHUNK):
                x_rdmas[c].wait_recv()
            for c in range(N_CHUNK):
                x_rdmas[c].wait_send()

    return pl.pallas_call(
        body,
        out_shape=jax.ShapeDtypeStruct((t, d), jnp.bfloat16),
        in_specs=[
            pl.BlockSpec(memory_space=pltpu.SMEM),
            pl.BlockSpec(memory_space=pltpu.VMEM),
            pl.BlockSpec(memory_space=pltpu.MemorySpace.HBM),
        ],
        out_specs=pl.BlockSpec(memory_space=pltpu.VMEM),
        scratch_shapes=[
            pltpu.VMEM((RING, rows, d), jnp.float32),
            pltpu.VMEM((half, d), jnp.bfloat16),
            pltpu.SemaphoreType.DMA((RING,)),
            pltpu.SemaphoreType.DMA((N_CHUNK,)),
            pltpu.SemaphoreType.DMA((N_CHUNK,)),
            pltpu.SemaphoreType.DMA((N_CHUNK,)),
            pltpu.SemaphoreType.DMA((N_CHUNK,)),
        ],
        compiler_params=pltpu.CompilerParams(
            collective_id=None if NOCOMM else 0,
            vmem_limit_bytes=56 * 1024 * 1024,
        ),
    )(idx, scale, E)


def kernel(ids, E):
    v_per = E.shape[0]
    my_y = lax.axis_index("y")
    local = ids - my_y * v_per
    mask = (local >= 0) & (local < v_per)
    idx = jnp.clip(local, 0, v_per - 1)
    scale = mask.astype(jnp.float32)[:, None]
    return _fused(idx, scale, E)
